# baseline (speedup 1.0000x reference)
import jax
import jax.numpy as jnp
from jax import lax
from jax.experimental import pallas as pl
from jax.experimental.pallas import tpu as pltpu

_BM = 400
_NBUF = 2


def _gcn_body(x_ref, w_ref, b_ref, adj_hbm, o_hbm,
              h_ref, bufs, obuf, in_sems, out_sems):
    n = x_ref.shape[0]
    nblk = n // _BM

    def in_copy(blk, slot):
        return pltpu.make_async_copy(
            adj_hbm.at[pl.ds(blk * _BM, _BM), :], bufs.at[slot],
            in_sems.at[slot])

    def out_copy(blk, slot):
        return pltpu.make_async_copy(
            obuf.at[slot], o_hbm.at[pl.ds(blk * _BM, _BM), :],
            out_sems.at[slot])

    for s in range(_NBUF):
        in_copy(s, s).start()

    h_ref[...] = jnp.dot(
        x_ref[...].astype(jnp.bfloat16),
        w_ref[...].astype(jnp.bfloat16),
        preferred_element_type=jnp.float32,
    ).astype(jnp.bfloat16)

    def step(i, carry):
        slot = lax.rem(i, _NBUF)
        in_copy(i, slot).wait()
        r = jnp.dot(
            bufs[slot].astype(jnp.bfloat16), h_ref[...],
            preferred_element_type=jnp.float32,
        ) + b_ref[...]
        oslot = lax.rem(i, 2)

        @pl.when(i >= 2)
        def _():
            out_copy(i - 2, oslot).wait()

        obuf[oslot] = r
        out_copy(i, oslot).start()

        @pl.when(i + _NBUF < nblk)
        def _():
            in_copy(i + _NBUF, slot).start()

        return carry

    lax.fori_loop(0, nblk, step, 0)
    out_copy(nblk - 2, (nblk - 2) % 2).wait()
    out_copy(nblk - 1, (nblk - 1) % 2).wait()


def kernel(input, adj, W, b):
    n, in_dim = input.shape
    out_dim = W.shape[1]
    b2 = b.reshape(1, out_dim)
    out = pl.pallas_call(
        _gcn_body,
        in_specs=[
            pl.BlockSpec((n, in_dim), lambda: (0, 0)),
            pl.BlockSpec((in_dim, out_dim), lambda: (0, 0)),
            pl.BlockSpec((1, out_dim), lambda: (0, 0)),
            pl.BlockSpec(memory_space=pltpu.HBM),
        ],
        out_specs=pl.BlockSpec(memory_space=pltpu.HBM),
        out_shape=jax.ShapeDtypeStruct((n, out_dim), jnp.float32),
        scratch_shapes=[
            pltpu.VMEM((n, out_dim), jnp.bfloat16),
            pltpu.VMEM((_NBUF, _BM, n), jnp.float32),
            pltpu.VMEM((2, _BM, out_dim), jnp.float32),
            pltpu.SemaphoreType.DMA((_NBUF,)),
            pltpu.SemaphoreType.DMA((2,)),
        ],
    )(input, W, b2, adj)
    return out


# speedup vs baseline: 1.0100x; 1.0100x over previous
import jax
import jax.numpy as jnp
from jax.experimental import pallas as pl
from jax.experimental.pallas import tpu as pltpu

_BM = 400


def _gcn_body(x_ref, w_ref, adj_ref, b_ref, o_ref, h_ref):
    @pl.when(pl.program_id(0) == 0)
    def _():
        xw = jnp.dot(
            x_ref[...].astype(jnp.bfloat16),
            w_ref[...].astype(jnp.bfloat16),
            preferred_element_type=jnp.float32,
        )
        h_ref[...] = xw.astype(jnp.bfloat16)

    a = adj_ref[...].astype(jnp.bfloat16)
    o_ref[...] = (
        jnp.dot(a, h_ref[...], preferred_element_type=jnp.float32) + b_ref[...]
    )


def kernel(input, adj, W, b):
    n, in_dim = input.shape
    out_dim = W.shape[1]
    bm = _BM if n % _BM == 0 else n
    grid = (n // bm,)
    b2 = b.reshape(1, out_dim)
    out = pl.pallas_call(
        _gcn_body,
        grid=grid,
        in_specs=[
            pl.BlockSpec((n, in_dim), lambda i: (0, 0)),
            pl.BlockSpec((in_dim, out_dim), lambda i: (0, 0)),
            pl.BlockSpec((bm, n), lambda i: (i, 0)),
            pl.BlockSpec((1, out_dim), lambda i: (0, 0)),
        ],
        out_specs=pl.BlockSpec((bm, out_dim), lambda i: (i, 0)),
        out_shape=jax.ShapeDtypeStruct((n, out_dim), jnp.float32),
        scratch_shapes=[pltpu.VMEM((n, out_dim), jnp.bfloat16)],
    )(input, W, adj, b2)
    return out
